# baseline (speedup 1.0000x reference)
import functools

import jax
import jax.numpy as jnp
from jax import lax
from jax.experimental import pallas as pl
from jax.experimental.pallas import tpu as pltpu
from jax.experimental.pallas import tpu_sc as plsc

D = 32
B = 4096 * 200

NC = 2
NS = 16
NW = NC * NS
B_PER_W = B // NW
NBUF = 4
CHUNK = 800
N_CHUNKS = B_PER_W // CHUNK


def _make_gather():
    mesh = plsc.VectorSubcoreMesh(core_axis_name="c", subcore_axis_name="s")

    scratch = (
        [pltpu.VMEM((B_PER_W,), jnp.int32)]
        + [pltpu.VMEM((CHUNK, D), jnp.float32) for _ in range(NBUF)]
        + [pltpu.SemaphoreType.DMA for _ in range(2 * NBUF)]
    )

    @functools.partial(
        pl.kernel,
        mesh=mesh,
        compiler_params=pltpu.CompilerParams(use_tc_tiling_on_sc=False),
        out_type=jax.ShapeDtypeStruct((B, D), jnp.float32),
        scratch_types=scratch,
    )
    def gather_k(idx_hbm, table_hbm, out_hbm, *refs):
        idx_v = refs[0]
        rows_v = refs[1:1 + NBUF]
        gsem = refs[1 + NBUF:1 + 2 * NBUF]
        osem = refs[1 + 2 * NBUF:1 + 3 * NBUF]

        wid = lax.axis_index("s") * NC + lax.axis_index("c")
        base = wid * B_PER_W

        pltpu.sync_copy(idx_hbm.at[pl.ds(base, B_PER_W)], idx_v)

        gather_d = [None] * N_CHUNKS
        out_d = [None] * N_CHUNKS
        for t in range(N_CHUNKS + 1):
            if t < N_CHUNKS:
                b = t % NBUF
                if t >= NBUF:
                    out_d[t - NBUF].wait()
                gather_d[t] = pltpu.async_copy(
                    table_hbm.at[idx_v.at[pl.ds(t * CHUNK, CHUNK)]],
                    rows_v[b], gsem[b])
            if t >= 1:
                g = t - 1
                b = g % NBUF
                gather_d[g].wait()
                off = base + g * CHUNK
                out_d[g] = pltpu.async_copy(
                    rows_v[b], out_hbm.at[pl.ds(off, CHUNK)], osem[b])
        for g in range(N_CHUNKS - NBUF, N_CHUNKS):
            out_d[g].wait()

    return gather_k


_gather = _make_gather()


def kernel(input_variable, table):
    idx = input_variable.reshape(B).astype(jnp.int32)
    out = _gather(idx, table)
    return out.reshape(input_variable.shape[0], input_variable.shape[1], D)
